# Initial kernel scaffold; baseline (speedup 1.0000x reference)
import jax
import jax.numpy as jnp
from jax.experimental import pallas as pl


def kernel(x, edge_index, W1, att_src1, att_dst1, b1, W2, att_src2, att_dst2, b2, fc_w, fc_b):
    raise NotImplementedError("write your pallas kernel here")


# speedup vs baseline: 40.8355x; 40.8355x over previous
import functools
import jax
import jax.numpy as jnp
from jax import lax
from jax.experimental import pallas as pl
from jax.experimental.pallas import tpu as pltpu
from jax.experimental.pallas import tpu_sc as plsc

N_NODES = 10000
NP = 10112
D = 128
NC = 2
NS = 16
LANES = 16
CHUNK = 128


def _dense_body(xp_ref, wt_ref, pa_ref, pb_ref, h_ref, as_ref, ad_ref):
    h = jnp.dot(xp_ref[...], wt_ref[...], preferred_element_type=jnp.float32)
    h_ref[...] = h
    as_ref[...] = jnp.dot(h, pa_ref[...], preferred_element_type=jnp.float32)
    ad_ref[...] = jnp.dot(h, pb_ref[...], preferred_element_type=jnp.float32)


def _dense(xp, wt, pa, pb):
    return pl.pallas_call(
        _dense_body,
        out_shape=[
            jax.ShapeDtypeStruct((NP, D), jnp.float32),
            jax.ShapeDtypeStruct((NP, 16), jnp.float32),
            jax.ShapeDtypeStruct((NP, 16), jnp.float32),
        ],
    )(xp, wt, pa, pb)


def _comb_body(acc_ref, s_ref, r_ref, b_ref, wt_ref, pa_ref, pb_ref,
               h_ref, as_ref, ad_ref):
    accsum = acc_ref[0] + acc_ref[1]
    ssum = s_ref[0] + s_ref[1]
    srep = jnp.dot(ssum, r_ref[...], preferred_element_type=jnp.float32)
    x2 = jnp.maximum(accsum / (srep + 1e-16) + b_ref[...], 0.0)
    h2 = jnp.dot(x2, wt_ref[...], preferred_element_type=jnp.float32)
    h_ref[...] = h2
    as_ref[...] = jnp.dot(h2, pa_ref[...], preferred_element_type=jnp.float32)
    ad_ref[...] = jnp.dot(h2, pb_ref[...], preferred_element_type=jnp.float32)


def _combine_dense(acc, s, r, b, wt, pa, pb):
    return pl.pallas_call(
        _comb_body,
        out_shape=[
            jax.ShapeDtypeStruct((NP, D), jnp.float32),
            jax.ShapeDtypeStruct((NP, 16), jnp.float32),
            jax.ShapeDtypeStruct((NP, 16), jnp.float32),
        ],
    )(acc, s, r, b, wt, pa, pb)


def _final_body(acc_ref, s_ref, r_ref, b_ref, fcw_ref, fcb_ref,
                emb_ref, out_ref):
    accsum = acc_ref[0] + acc_ref[1]
    ssum = s_ref[0] + s_ref[1]
    srep = jnp.dot(ssum, r_ref[...], preferred_element_type=jnp.float32)
    emb = jnp.maximum(accsum / (srep + 1e-16) + b_ref[...], 0.0)
    emb_ref[...] = emb
    out_ref[...] = jnp.dot(emb, fcw_ref[...],
                           preferred_element_type=jnp.float32) + fcb_ref[...]


def _final(acc, s, r, b, fcw, fcb):
    return pl.pallas_call(
        _final_body,
        out_shape=[
            jax.ShapeDtypeStruct((NP, D), jnp.float32),
            jax.ShapeDtypeStruct((NP, 64), jnp.float32),
        ],
    )(acc, s, r, b, fcw, fcb)


def _edge_body(e_pad, src_ref, dst_ref, h_hbm, asrc_hbm, adst_hbm,
               z128_hbm, z16_hbm, acc_out, s_out,
               sidx, didx, av, bv, wbuf, hv, acc_sh, s_sh):
    cid = lax.axis_index("c")
    sid = lax.axis_index("s")
    wid = cid * NS + sid
    rpt = NP // NS
    r0 = sid * rpt
    pltpu.sync_copy(z128_hbm.at[pl.ds(r0, rpt)], acc_sh.at[pl.ds(r0, rpt)])
    pltpu.sync_copy(z16_hbm.at[pl.ds(r0, rpt)], s_sh.at[pl.ds(r0, rpt)])
    plsc.subcore_barrier()

    ept = e_pad // (NC * NS)
    nchunks = ept // CHUNK
    base0 = wid * ept

    def chunk(i, carry):
        base = base0 + i * CHUNK
        pltpu.sync_copy(src_ref.at[pl.ds(base, CHUNK)], sidx)
        pltpu.sync_copy(dst_ref.at[pl.ds(base, CHUNK)], didx)
        pltpu.sync_copy(asrc_hbm.at[sidx], av)
        pltpu.sync_copy(adst_hbm.at[didx], bv)
        pltpu.sync_copy(h_hbm.at[sidx], hv)

        def att(j, c):
            v = av[j] + bv[j]
            wbuf[j] = jnp.exp(jnp.maximum(v, 0.2 * v))
            return c
        lax.fori_loop(0, CHUNK, att, 0)

        def mul(k, c):
            wrow = wbuf[k]
            for hh in range(8):
                sl = pl.ds(hh * LANES, LANES)
                hv[k, sl] = hv[k, sl] * wrow[hh]
            return c
        lax.fori_loop(0, CHUNK, mul, 0)

        pltpu.sync_copy(wbuf, s_sh.at[didx], add=True)
        pltpu.sync_copy(hv, acc_sh.at[didx], add=True)
        return carry

    lax.fori_loop(0, nchunks, chunk, 0)
    plsc.subcore_barrier()
    pltpu.sync_copy(acc_sh.at[pl.ds(r0, rpt)],
                    acc_out.at[cid, pl.ds(r0, rpt)])
    pltpu.sync_copy(s_sh.at[pl.ds(r0, rpt)],
                    s_out.at[cid, pl.ds(r0, rpt)])


def _edge_pass(e_pad, src_p, dst_p, h, asrc, adst, z128, z16):
    mesh = plsc.VectorSubcoreMesh(core_axis_name="c", subcore_axis_name="s")
    f = pl.kernel(
        functools.partial(_edge_body, e_pad),
        out_type=[
            jax.ShapeDtypeStruct((NC, NP, D), jnp.float32),
            jax.ShapeDtypeStruct((NC, NP, 16), jnp.float32),
        ],
        mesh=mesh,
        compiler_params=pltpu.CompilerParams(use_tc_tiling_on_sc=False),
        scratch_types=[
            pltpu.VMEM((CHUNK,), jnp.int32),
            pltpu.VMEM((CHUNK,), jnp.int32),
            pltpu.VMEM((CHUNK, 16), jnp.float32),
            pltpu.VMEM((CHUNK, 16), jnp.float32),
            pltpu.VMEM((CHUNK, 16), jnp.float32),
            pltpu.VMEM((CHUNK, D), jnp.float32),
            pltpu.VMEM_SHARED((NP, D), jnp.float32),
            pltpu.VMEM_SHARED((NP, 16), jnp.float32),
        ],
    )
    return f(src_p, dst_p, h, asrc, adst, z128, z16)


def _att_proj_mat(att):
    heads, ch = att.shape
    if heads == 1:
        return jnp.broadcast_to(att.reshape(D, 1), (D, 16))
    m = (att[:, :, None] * jnp.eye(heads, dtype=att.dtype)[:, None, :])
    m = m.reshape(heads * ch, heads)
    return jnp.pad(m, ((0, 0), (0, 16 - heads)))


def _srep_mat(heads):
    if heads == 1:
        r = jnp.zeros((16, D), jnp.float32).at[0, :].set(1.0)
        return r
    r = jnp.kron(jnp.eye(heads, dtype=jnp.float32),
                 jnp.ones((1, D // heads), jnp.float32))
    return jnp.pad(r, ((0, 16 - heads), (0, 0)))


def kernel(x, edge_index, W1, att_src1, att_dst1, b1,
           W2, att_src2, att_dst2, b2, fc_w, fc_b):
    n = x.shape[0]
    loops = jnp.arange(n, dtype=jnp.int32)
    src = jnp.concatenate([edge_index[0].astype(jnp.int32), loops])
    dst = jnp.concatenate([edge_index[1].astype(jnp.int32), loops])
    e_real = src.shape[0]
    grain = NC * NS * CHUNK
    e_pad = grain * ((e_real + grain - 1) // grain)
    src_p = jnp.zeros((e_pad,), jnp.int32).at[:e_real].set(src)
    dst_p = jnp.full((e_pad,), n, jnp.int32).at[:e_real].set(dst)

    xp = jnp.pad(x, ((0, NP - n), (0, 0)))
    z128 = jnp.zeros((NP, D), jnp.float32)
    z16 = jnp.zeros((NP, 16), jnp.float32)

    pa1 = _att_proj_mat(att_src1)
    pb1 = _att_proj_mat(att_dst1)
    pa2 = _att_proj_mat(att_src2)
    pb2 = _att_proj_mat(att_dst2)
    r1 = _srep_mat(att_src1.shape[0])
    r2 = _srep_mat(att_src2.shape[0])

    h1, as1, ad1 = _dense(xp, W1.T, pa1, pb1)
    acc1, s1 = _edge_pass(e_pad, src_p, dst_p, h1, as1, ad1, z128, z16)
    h2, as2, ad2 = _combine_dense(acc1, s1, r1, b1[None, :], W2.T, pa2, pb2)
    acc2, s2 = _edge_pass(e_pad, src_p, dst_p, h2, as2, ad2, z128, z16)
    emb, out = _final(acc2, s2, r2, b2[None, :], fc_w.T, fc_b[None, :])
    return (emb[:n], out[:n])


# speedup vs baseline: 62.5275x; 1.5312x over previous
import functools
import jax
import jax.numpy as jnp
from jax import lax
from jax.experimental import pallas as pl
from jax.experimental.pallas import tpu as pltpu
from jax.experimental.pallas import tpu_sc as plsc

N_NODES = 10000
NP = 10112
D = 128
NC = 2
NS = 16
LANES = 16
CHUNK = 64


def _dense_body(xp_ref, wt_ref, pa_ref, pb_ref, h_ref, as_ref, ad_ref):
    h = jnp.dot(xp_ref[...], wt_ref[...], preferred_element_type=jnp.float32)
    h_ref[...] = h
    as_ref[...] = jnp.dot(h, pa_ref[...], preferred_element_type=jnp.float32)
    ad_ref[...] = jnp.dot(h, pb_ref[...], preferred_element_type=jnp.float32)


def _dense(xp, wt, pa, pb):
    return pl.pallas_call(
        _dense_body,
        out_shape=[
            jax.ShapeDtypeStruct((NP, D), jnp.float32),
            jax.ShapeDtypeStruct((NP, 16), jnp.float32),
            jax.ShapeDtypeStruct((NP, 16), jnp.float32),
        ],
    )(xp, wt, pa, pb)


def _comb_body(acc_ref, s_ref, r_ref, b_ref, wt_ref, pa_ref, pb_ref,
               h_ref, as_ref, ad_ref):
    accsum = acc_ref[0] + acc_ref[1]
    ssum = s_ref[0] + s_ref[1]
    srep = jnp.dot(ssum, r_ref[...], preferred_element_type=jnp.float32)
    x2 = jnp.maximum(accsum / (srep + 1e-16) + b_ref[...], 0.0)
    h2 = jnp.dot(x2, wt_ref[...], preferred_element_type=jnp.float32)
    h_ref[...] = h2
    as_ref[...] = jnp.dot(h2, pa_ref[...], preferred_element_type=jnp.float32)
    ad_ref[...] = jnp.dot(h2, pb_ref[...], preferred_element_type=jnp.float32)


def _combine_dense(acc, s, r, b, wt, pa, pb):
    return pl.pallas_call(
        _comb_body,
        out_shape=[
            jax.ShapeDtypeStruct((NP, D), jnp.float32),
            jax.ShapeDtypeStruct((NP, 16), jnp.float32),
            jax.ShapeDtypeStruct((NP, 16), jnp.float32),
        ],
    )(acc, s, r, b, wt, pa, pb)


def _final_body(acc_ref, s_ref, r_ref, b_ref, fcw_ref, fcb_ref,
                emb_ref, out_ref):
    accsum = acc_ref[0] + acc_ref[1]
    ssum = s_ref[0] + s_ref[1]
    srep = jnp.dot(ssum, r_ref[...], preferred_element_type=jnp.float32)
    emb = jnp.maximum(accsum / (srep + 1e-16) + b_ref[...], 0.0)
    emb_ref[...] = emb
    out_ref[...] = jnp.dot(emb, fcw_ref[...],
                           preferred_element_type=jnp.float32) + fcb_ref[...]


def _final(acc, s, r, b, fcw, fcb):
    return pl.pallas_call(
        _final_body,
        out_shape=[
            jax.ShapeDtypeStruct((NP, D), jnp.float32),
            jax.ShapeDtypeStruct((NP, 64), jnp.float32),
        ],
    )(acc, s, r, b, fcw, fcb)


NBUF = 3


def _edge_body(e_pad, src_ref, dst_ref, h_hbm, asrc_hbm, adst_hbm,
               z128_hbm, z16_hbm, acc_out, s_out,
               sidx_w, didx_w, av, bv, wbuf, hv,
               acc_sh, s_sh, sem_i, sem_g, sem_s):
    cid = lax.axis_index("c")
    sid = lax.axis_index("s")
    wid = cid * NS + sid
    rpt = NP // NS
    r0 = sid * rpt
    pltpu.sync_copy(z128_hbm.at[pl.ds(r0, rpt)], acc_sh.at[pl.ds(r0, rpt)])
    pltpu.sync_copy(z16_hbm.at[pl.ds(r0, rpt)], s_sh.at[pl.ds(r0, rpt)])

    ept = e_pad // (NC * NS)
    nchunks = ept // CHUNK
    base0 = wid * ept
    plsc.subcore_barrier()

    def idx_copies(i, b):
        base = base0 + i * CHUNK
        return [
            pltpu.make_async_copy(src_ref.at[pl.ds(base, CHUNK)],
                                  sidx_w.at[b], sem_i.at[b]),
            pltpu.make_async_copy(dst_ref.at[pl.ds(base, CHUNK)],
                                  didx_w.at[b], sem_i.at[b]),
        ]

    def gather_copies(b):
        return [
            pltpu.make_async_copy(asrc_hbm.at[sidx_w.at[b]],
                                  av.at[b], sem_g.at[b]),
            pltpu.make_async_copy(adst_hbm.at[didx_w.at[b]],
                                  bv.at[b], sem_g.at[b]),
            pltpu.make_async_copy(h_hbm.at[sidx_w.at[b]],
                                  hv.at[b], sem_g.at[b]),
        ]

    def scatter_copies(b):
        return [
            pltpu.make_async_copy(wbuf.at[b], s_sh.at[didx_w.at[b]],
                                  sem_s.at[b]),
            pltpu.make_async_copy(hv.at[b], acc_sh.at[didx_w.at[b]],
                                  sem_s.at[b]),
        ]

    def compute(b):
        avb, bvb, wb, hvb = av.at[b], bv.at[b], wbuf.at[b], hv.at[b]

        def att(j, c):
            v = avb[j] + bvb[j]
            wb[j] = jnp.exp(jnp.maximum(v, 0.2 * v))
            return c
        lax.fori_loop(0, CHUNK, att, 0, unroll=4)

        def mul(k, c):
            wrow = wb[k]
            for hh in range(8):
                sl = pl.ds(hh * LANES, LANES)
                hvb[k, sl] = hvb[k, sl] * wrow[hh]
            return c
        lax.fori_loop(0, CHUNK, mul, 0, unroll=2)

    for c in idx_copies(0, 0):
        c.start()
    for c in idx_copies(1, 1):
        c.start()
    for c in idx_copies(0, 0):
        c.wait()
    for c in gather_copies(0):
        c.start()

    def group(g, carry):
        for b in range(NBUF):
            i = g * NBUF + b
            pb = (b + 2) % NBUF
            b1 = (b + 1) % NBUF

            @pl.when(i >= 1)
            def _():
                for c in scatter_copies(pb):
                    c.wait()

            @pl.when(i + 2 < nchunks)
            def _():
                for c in idx_copies(i + 2, pb):
                    c.start()

            @pl.when(i + 1 < nchunks)
            def _():
                for c in idx_copies(i + 1, b1):
                    c.wait()
                for c in gather_copies(b1):
                    c.start()

            for c in gather_copies(b):
                c.wait()
            compute(b)
            for c in scatter_copies(b):
                c.start()
        return carry

    lax.fori_loop(0, nchunks // NBUF, group, 0)
    for c in scatter_copies((nchunks - 1) % NBUF):
        c.wait()
    plsc.subcore_barrier()
    pltpu.sync_copy(acc_sh.at[pl.ds(r0, rpt)],
                    acc_out.at[cid, pl.ds(r0, rpt)])
    pltpu.sync_copy(s_sh.at[pl.ds(r0, rpt)],
                    s_out.at[cid, pl.ds(r0, rpt)])


def _edge_pass(e_pad, src_p, dst_p, h, asrc, adst, z128, z16):
    ept = e_pad // (NC * NS)
    mesh = plsc.VectorSubcoreMesh(core_axis_name="c", subcore_axis_name="s")
    f = pl.kernel(
        functools.partial(_edge_body, e_pad),
        out_type=[
            jax.ShapeDtypeStruct((NC, NP, D), jnp.float32),
            jax.ShapeDtypeStruct((NC, NP, 16), jnp.float32),
        ],
        mesh=mesh,
        compiler_params=pltpu.CompilerParams(use_tc_tiling_on_sc=False),
        scratch_types=[
            pltpu.VMEM((NBUF, CHUNK), jnp.int32),
            pltpu.VMEM((NBUF, CHUNK), jnp.int32),
            pltpu.VMEM((NBUF, CHUNK, 16), jnp.float32),
            pltpu.VMEM((NBUF, CHUNK, 16), jnp.float32),
            pltpu.VMEM((NBUF, CHUNK, 16), jnp.float32),
            pltpu.VMEM((NBUF, CHUNK, D), jnp.float32),
            pltpu.VMEM_SHARED((NP, D), jnp.float32),
            pltpu.VMEM_SHARED((NP, 16), jnp.float32),
            pltpu.SemaphoreType.DMA((NBUF,)),
            pltpu.SemaphoreType.DMA((NBUF,)),
            pltpu.SemaphoreType.DMA((NBUF,)),
        ],
    )
    return f(src_p, dst_p, h, asrc, adst, z128, z16)


def _att_proj_mat(att):
    heads, ch = att.shape
    if heads == 1:
        return jnp.broadcast_to(att.reshape(D, 1), (D, 16))
    m = (att[:, :, None] * jnp.eye(heads, dtype=att.dtype)[:, None, :])
    m = m.reshape(heads * ch, heads)
    return jnp.pad(m, ((0, 0), (0, 16 - heads)))


def _srep_mat(heads):
    if heads == 1:
        r = jnp.zeros((16, D), jnp.float32).at[0, :].set(1.0)
        return r
    r = jnp.kron(jnp.eye(heads, dtype=jnp.float32),
                 jnp.ones((1, D // heads), jnp.float32))
    return jnp.pad(r, ((0, 16 - heads), (0, 0)))


def kernel(x, edge_index, W1, att_src1, att_dst1, b1,
           W2, att_src2, att_dst2, b2, fc_w, fc_b):
    n = x.shape[0]
    loops = jnp.arange(n, dtype=jnp.int32)
    src = jnp.concatenate([edge_index[0].astype(jnp.int32), loops])
    dst = jnp.concatenate([edge_index[1].astype(jnp.int32), loops])
    e_real = src.shape[0]
    grain = NC * NS * CHUNK * NBUF
    e_pad = grain * ((e_real + grain - 1) // grain)
    src_p = jnp.zeros((e_pad,), jnp.int32).at[:e_real].set(src)
    dst_p = jnp.full((e_pad,), n, jnp.int32).at[:e_real].set(dst)

    xp = jnp.pad(x, ((0, NP - n), (0, 0)))
    z128 = jnp.zeros((NP, D), jnp.float32)
    z16 = jnp.zeros((NP, 16), jnp.float32)

    pa1 = _att_proj_mat(att_src1)
    pb1 = _att_proj_mat(att_dst1)
    pa2 = _att_proj_mat(att_src2)
    pb2 = _att_proj_mat(att_dst2)
    r1 = _srep_mat(att_src1.shape[0])
    r2 = _srep_mat(att_src2.shape[0])

    h1, as1, ad1 = _dense(xp, W1.T, pa1, pb1)
    acc1, s1 = _edge_pass(e_pad, src_p, dst_p, h1, as1, ad1, z128, z16)
    h2, as2, ad2 = _combine_dense(acc1, s1, r1, b1[None, :], W2.T, pa2, pb2)
    acc2, s2 = _edge_pass(e_pad, src_p, dst_p, h2, as2, ad2, z128, z16)
    emb, out = _final(acc2, s2, r2, b2[None, :], fc_w.T, fc_b[None, :])
    return (emb[:n], out[:n])


# speedup vs baseline: 62.5781x; 1.0008x over previous
import functools
import jax
import jax.numpy as jnp
from jax import lax
from jax.experimental import pallas as pl
from jax.experimental.pallas import tpu as pltpu
from jax.experimental.pallas import tpu_sc as plsc

N_NODES = 10000
NP = 10112
D = 128
NC = 2
NS = 16
LANES = 16
CHUNK = 64


def _dense_body(xp_ref, wt_ref, pa_ref, pb_ref, h_ref, as_ref, ad_ref):
    h = jnp.dot(xp_ref[...], wt_ref[...], preferred_element_type=jnp.float32)
    h_ref[...] = h
    as_ref[...] = jnp.dot(h, pa_ref[...], preferred_element_type=jnp.float32)
    ad_ref[...] = jnp.dot(h, pb_ref[...], preferred_element_type=jnp.float32)


def _dense(xp, wt, pa, pb):
    return pl.pallas_call(
        _dense_body,
        out_shape=[
            jax.ShapeDtypeStruct((NP, D), jnp.float32),
            jax.ShapeDtypeStruct((NP, 16), jnp.float32),
            jax.ShapeDtypeStruct((NP, 16), jnp.float32),
        ],
    )(xp, wt, pa, pb)


def _comb_body(acc_ref, s_ref, r_ref, b_ref, wt_ref, pa_ref, pb_ref,
               h_ref, as_ref, ad_ref):
    accsum = acc_ref[0] + acc_ref[1]
    ssum = s_ref[0] + s_ref[1]
    srep = jnp.dot(ssum, r_ref[...], preferred_element_type=jnp.float32)
    x2 = jnp.maximum(accsum / (srep + 1e-16) + b_ref[...], 0.0)
    h2 = jnp.dot(x2, wt_ref[...], preferred_element_type=jnp.float32)
    h_ref[...] = h2
    as_ref[...] = jnp.dot(h2, pa_ref[...], preferred_element_type=jnp.float32)
    ad_ref[...] = jnp.dot(h2, pb_ref[...], preferred_element_type=jnp.float32)


def _combine_dense(acc, s, r, b, wt, pa, pb):
    return pl.pallas_call(
        _comb_body,
        out_shape=[
            jax.ShapeDtypeStruct((NP, D), jnp.float32),
            jax.ShapeDtypeStruct((NP, 16), jnp.float32),
            jax.ShapeDtypeStruct((NP, 16), jnp.float32),
        ],
    )(acc, s, r, b, wt, pa, pb)


def _final_body(acc_ref, s_ref, r_ref, b_ref, fcw_ref, fcb_ref,
                emb_ref, out_ref):
    accsum = acc_ref[0] + acc_ref[1]
    ssum = s_ref[0] + s_ref[1]
    srep = jnp.dot(ssum, r_ref[...], preferred_element_type=jnp.float32)
    emb = jnp.maximum(accsum / (srep + 1e-16) + b_ref[...], 0.0)
    emb_ref[...] = emb
    out_ref[...] = jnp.dot(emb, fcw_ref[...],
                           preferred_element_type=jnp.float32) + fcb_ref[...]


def _final(acc, s, r, b, fcw, fcb):
    return pl.pallas_call(
        _final_body,
        out_shape=[
            jax.ShapeDtypeStruct((NP, D), jnp.float32),
            jax.ShapeDtypeStruct((NP, 64), jnp.float32),
        ],
    )(acc, s, r, b, fcw, fcb)


NBUF = 3


def _edge_body(e_pad, src_ref, dst_ref, h_hbm, asrc_hbm, adst_hbm,
               z128_hbm, z16_hbm, acc_out, s_out,
               sidx_w, didx_w, av, bv, wbuf, hv,
               acc_sh, s_sh, sem_i, sem_g, sem_s):
    cid = lax.axis_index("c")
    sid = lax.axis_index("s")
    wid = cid * NS + sid
    rpt = NP // NS
    r0 = sid * rpt
    pltpu.sync_copy(z128_hbm.at[pl.ds(r0, rpt)], acc_sh.at[pl.ds(r0, rpt)])
    pltpu.sync_copy(z16_hbm.at[pl.ds(r0, rpt)], s_sh.at[pl.ds(r0, rpt)])

    ept = e_pad // (NC * NS)
    nchunks = ept // CHUNK
    base0 = wid * ept
    plsc.subcore_barrier()

    def idx_copies(i, b):
        base = base0 + i * CHUNK
        return [
            pltpu.make_async_copy(src_ref.at[pl.ds(base, CHUNK)],
                                  sidx_w.at[b], sem_i.at[b]),
            pltpu.make_async_copy(dst_ref.at[pl.ds(base, CHUNK)],
                                  didx_w.at[b], sem_i.at[b]),
        ]

    def gather_copies(b):
        return [
            pltpu.make_async_copy(asrc_hbm.at[sidx_w.at[b]],
                                  av.at[b], sem_g.at[b]),
            pltpu.make_async_copy(adst_hbm.at[didx_w.at[b]],
                                  bv.at[b], sem_g.at[b]),
            pltpu.make_async_copy(h_hbm.at[sidx_w.at[b]],
                                  hv.at[b], sem_g.at[b]),
        ]

    def scatter_copies(b):
        return [
            pltpu.make_async_copy(wbuf.at[b], s_sh.at[didx_w.at[b]],
                                  sem_s.at[b]),
            pltpu.make_async_copy(hv.at[b], acc_sh.at[didx_w.at[b]],
                                  sem_s.at[b]),
        ]

    def compute(b):
        avb, bvb, wb, hvb = av.at[b], bv.at[b], wbuf.at[b], hv.at[b]

        def att(j, c):
            v = avb[j] + bvb[j]
            wb[j] = jnp.exp(jnp.maximum(v, 0.2 * v))
            return c
        lax.fori_loop(0, CHUNK, att, 0, unroll=4)

        def mul(k, c):
            wrow = wb[k]
            for hh in range(8):
                sl = pl.ds(hh * LANES, LANES)
                hvb[k, sl] = hvb[k, sl] * wrow[hh]
            return c
        lax.fori_loop(0, CHUNK, mul, 0, unroll=2)

    for c in idx_copies(0, 0):
        c.start()
    for c in idx_copies(1, 1):
        c.start()
    for c in idx_copies(0, 0):
        c.wait()
    for c in gather_copies(0):
        c.start()

    def group(g, carry):
        for b in range(NBUF):
            i = g * NBUF + b
            pb = (b + 2) % NBUF
            b1 = (b + 1) % NBUF

            @pl.when(i >= 1)
            def _():
                for c in scatter_copies(pb):
                    c.wait()

            @pl.when(i + 2 < nchunks)
            def _():
                for c in idx_copies(i + 2, pb):
                    c.start()

            @pl.when(i + 1 < nchunks)
            def _():
                for c in idx_copies(i + 1, b1):
                    c.wait()
                for c in gather_copies(b1):
                    c.start()

            for c in gather_copies(b):
                c.wait()
            compute(b)
            for c in scatter_copies(b):
                c.start(add=True)
        return carry

    lax.fori_loop(0, nchunks // NBUF, group, 0)
    for c in scatter_copies((nchunks - 1) % NBUF):
        c.wait()
    plsc.subcore_barrier()
    pltpu.sync_copy(acc_sh.at[pl.ds(r0, rpt)],
                    acc_out.at[cid, pl.ds(r0, rpt)])
    pltpu.sync_copy(s_sh.at[pl.ds(r0, rpt)],
                    s_out.at[cid, pl.ds(r0, rpt)])


def _edge_pass(e_pad, src_p, dst_p, h, asrc, adst, z128, z16):
    ept = e_pad // (NC * NS)
    mesh = plsc.VectorSubcoreMesh(core_axis_name="c", subcore_axis_name="s")
    f = pl.kernel(
        functools.partial(_edge_body, e_pad),
        out_type=[
            jax.ShapeDtypeStruct((NC, NP, D), jnp.float32),
            jax.ShapeDtypeStruct((NC, NP, 16), jnp.float32),
        ],
        mesh=mesh,
        compiler_params=pltpu.CompilerParams(use_tc_tiling_on_sc=False),
        scratch_types=[
            pltpu.VMEM((NBUF, CHUNK), jnp.int32),
            pltpu.VMEM((NBUF, CHUNK), jnp.int32),
            pltpu.VMEM((NBUF, CHUNK, 16), jnp.float32),
            pltpu.VMEM((NBUF, CHUNK, 16), jnp.float32),
            pltpu.VMEM((NBUF, CHUNK, 16), jnp.float32),
            pltpu.VMEM((NBUF, CHUNK, D), jnp.float32),
            pltpu.VMEM_SHARED((NP, D), jnp.float32),
            pltpu.VMEM_SHARED((NP, 16), jnp.float32),
            pltpu.SemaphoreType.DMA((NBUF,)),
            pltpu.SemaphoreType.DMA((NBUF,)),
            pltpu.SemaphoreType.DMA((NBUF,)),
        ],
    )
    return f(src_p, dst_p, h, asrc, adst, z128, z16)


def _att_proj_mat(att):
    heads, ch = att.shape
    if heads == 1:
        return jnp.broadcast_to(att.reshape(D, 1), (D, 16))
    m = (att[:, :, None] * jnp.eye(heads, dtype=att.dtype)[:, None, :])
    m = m.reshape(heads * ch, heads)
    return jnp.pad(m, ((0, 0), (0, 16 - heads)))


def _srep_mat(heads):
    if heads == 1:
        r = jnp.zeros((16, D), jnp.float32).at[0, :].set(1.0)
        return r
    r = jnp.kron(jnp.eye(heads, dtype=jnp.float32),
                 jnp.ones((1, D // heads), jnp.float32))
    return jnp.pad(r, ((0, 16 - heads), (0, 0)))


def kernel(x, edge_index, W1, att_src1, att_dst1, b1,
           W2, att_src2, att_dst2, b2, fc_w, fc_b):
    n = x.shape[0]
    loops = jnp.arange(n, dtype=jnp.int32)
    src = jnp.concatenate([edge_index[0].astype(jnp.int32), loops])
    dst = jnp.concatenate([edge_index[1].astype(jnp.int32), loops])
    e_real = src.shape[0]
    grain = NC * NS * CHUNK * NBUF
    e_pad = grain * ((e_real + grain - 1) // grain)
    src_p = jnp.zeros((e_pad,), jnp.int32).at[:e_real].set(src)
    dst_p = jnp.full((e_pad,), n, jnp.int32).at[:e_real].set(dst)

    xp = jnp.pad(x, ((0, NP - n), (0, 0)))
    z128 = jnp.zeros((NP, D), jnp.float32)
    z16 = jnp.zeros((NP, 16), jnp.float32)

    pa1 = _att_proj_mat(att_src1)
    pb1 = _att_proj_mat(att_dst1)
    pa2 = _att_proj_mat(att_src2)
    pb2 = _att_proj_mat(att_dst2)
    r1 = _srep_mat(att_src1.shape[0])
    r2 = _srep_mat(att_src2.shape[0])

    h1, as1, ad1 = _dense(xp, W1.T, pa1, pb1)
    acc1, s1 = _edge_pass(e_pad, src_p, dst_p, h1, as1, ad1, z128, z16)
    h2, as2, ad2 = _combine_dense(acc1, s1, r1, b1[None, :], W2.T, pa2, pb2)
    acc2, s2 = _edge_pass(e_pad, src_p, dst_p, h2, as2, ad2, z128, z16)
    emb, out = _final(acc2, s2, r2, b2[None, :], fc_w.T, fc_b[None, :])
    return (emb[:n], out[:n])
